# baseline (speedup 1.0000x reference)
import functools

import jax
import jax.numpy as jnp
from jax import lax
from jax.experimental import pallas as pl
from jax.experimental.pallas import tpu as pltpu
from jax.experimental.pallas import tpu_sc as plsc

N = 10000
E = 320000
D = 128
G = 64
OUT = 2001
OUT_PAD = 2048

NC, NS = 2, 16
NW = NC * NS
EW = E // NW
CHUNK = 128
IB = 16
NSTAGE = 5
NCHUNK = NSTAGE * IB
EWP = NCHUNK * CHUNK
RPT = 632
NPAD = RPT * NS
RLAST = N - (NS - 1) * RPT
BN = 1000

_MESH = dict(core_axis_name="c", subcore_axis_name="s", num_cores=NC,
             num_subcores=NS)


def _sc_deg(dstw):
    mesh = plsc.VectorSubcoreMesh(**_MESH)

    @functools.partial(
        pl.kernel,
        out_type=jax.ShapeDtypeStruct((NC, N, D), jnp.float32),
        mesh=mesh,
        scratch_types=[
            pltpu.VMEM((NCHUNK, CHUNK), jnp.int32),
            pltpu.VMEM((CHUNK, D), jnp.float32),
            pltpu.VMEM((CHUNK, D), jnp.float32),
            pltpu.VMEM_SHARED((NPAD, D), jnp.float32),
        ],
    )
    def k(dstw_hbm, out_hbm, dst_v, ones_v, z_v, accum):
        c = lax.axis_index("c")
        s = lax.axis_index("s")
        r0 = pl.multiple_of(s * RPT, 8)

        pltpu.sync_copy(dstw_hbm.at[c, s], dst_v)

        one_row = jnp.where(lax.iota(jnp.int32, 16) == 0, 1.0, 0.0)
        zero_row = jnp.zeros((16,), jnp.float32)

        def fill_ones(i, _):
            for u in range(D // 16):
                ones_v[i, pl.ds(u * 16, 16)] = one_row if u == 0 else zero_row
                z_v[i, pl.ds(u * 16, 16)] = zero_row
            return 0

        lax.fori_loop(0, CHUNK, fill_ones, 0)

        for q in range(4):
            pltpu.sync_copy(z_v, accum.at[pl.ds(r0 + q * CHUNK, CHUNK)])

        @pl.when(s < NS - 1)
        def _():
            pltpu.sync_copy(z_v.at[pl.ds(0, RPT - 4 * CHUNK)],
                            accum.at[pl.ds(r0 + 4 * CHUNK, RPT - 4 * CHUNK)])

        @pl.when(s == NS - 1)
        def _():
            pltpu.sync_copy(
                z_v.at[pl.ds(0, RLAST - 4 * CHUNK)],
                accum.at[pl.ds((NS - 1) * RPT + 4 * CHUNK,
                               RLAST - 4 * CHUNK)])

        plsc.subcore_barrier()

        def body(j, _):
            pltpu.sync_copy(ones_v, accum.at[dst_v.at[j]], add=True)
            return 0

        lax.fori_loop(0, NCHUNK, body, 0)
        plsc.subcore_barrier()

        @pl.when(s < NS - 1)
        def _():
            pltpu.sync_copy(accum.at[pl.ds(r0, RPT)],
                            out_hbm.at[c, pl.ds(r0, RPT)])

        @pl.when(s == NS - 1)
        def _():
            pltpu.sync_copy(accum.at[pl.ds((NS - 1) * RPT, RLAST)],
                            out_hbm.at[c, pl.ds((NS - 1) * RPT, RLAST)])

    return k(dstw)


def _sc_agg(y, srcw, dstw):
    mesh = plsc.VectorSubcoreMesh(**_MESH)

    @functools.partial(
        pl.kernel,
        out_type=jax.ShapeDtypeStruct((NC, N, D), jnp.float32),
        mesh=mesh,
        scratch_types=[
            pltpu.VMEM((NCHUNK, CHUNK), jnp.int32),
            pltpu.VMEM((NCHUNK, CHUNK), jnp.int32),
            pltpu.VMEM((CHUNK, D), jnp.float32),
            pltpu.VMEM_SHARED((NPAD, D), jnp.float32),
            pltpu.SemaphoreType.DMA,
        ],
    )
    def k(y_hbm, srcw_hbm, dstw_hbm, out_hbm, src_v, dst_v, gbuf, accum,
          sem_g):
        c = lax.axis_index("c")
        s = lax.axis_index("s")
        r0 = pl.multiple_of(s * RPT, 8)

        pltpu.sync_copy(srcw_hbm.at[c, s], src_v)
        pltpu.sync_copy(dstw_hbm.at[c, s], dst_v)

        @pl.when(jnp.logical_and(c == 0, s < NS - 1))
        def _():
            pltpu.sync_copy(y_hbm.at[pl.ds(r0, RPT)], accum.at[pl.ds(r0, RPT)])

        @pl.when(jnp.logical_and(c == 0, s == NS - 1))
        def _():
            pltpu.sync_copy(y_hbm.at[pl.ds((NS - 1) * RPT, RLAST)],
                            accum.at[pl.ds((NS - 1) * RPT, RLAST)])

        @pl.when(c == 1)
        def _():
            def fill_zeros(i, _):
                for u in range(D // 16):
                    gbuf[i, pl.ds(u * 16, 16)] = jnp.zeros((16,),
                                                           jnp.float32)
                return 0

            lax.fori_loop(0, CHUNK, fill_zeros, 0)
            for q in range(4):
                pltpu.sync_copy(gbuf,
                                accum.at[pl.ds(r0 + q * CHUNK, CHUNK)])

            @pl.when(s < NS - 1)
            def _():
                pltpu.sync_copy(
                    gbuf.at[pl.ds(0, RPT - 4 * CHUNK)],
                    accum.at[pl.ds(r0 + 4 * CHUNK, RPT - 4 * CHUNK)])

            @pl.when(s == NS - 1)
            def _():
                pltpu.sync_copy(
                    gbuf.at[pl.ds(0, RLAST - 4 * CHUNK)],
                    accum.at[pl.ds((NS - 1) * RPT + 4 * CHUNK,
                                   RLAST - 4 * CHUNK)])

        plsc.subcore_barrier()

        def body(j, _):
            pltpu.async_copy(y_hbm.at[src_v.at[j]], gbuf, sem_g).wait()
            pltpu.sync_copy(gbuf, accum.at[dst_v.at[j]], add=True)
            return 0

        lax.fori_loop(0, NCHUNK, body, 0)
        plsc.subcore_barrier()

        @pl.when(s < NS - 1)
        def _():
            pltpu.sync_copy(accum.at[pl.ds(r0, RPT)],
                            out_hbm.at[c, pl.ds(r0, RPT)])

        @pl.when(s == NS - 1)
        def _():
            pltpu.sync_copy(accum.at[pl.ds((NS - 1) * RPT, RLAST)],
                            out_hbm.at[c, pl.ds((NS - 1) * RPT, RLAST)])

    return k(y, srcw, dstw)


def _tc1(degp, x, W1):
    def body(d0_ref, d1_ref, x_ref, w_ref, dis_ref, y_ref):
        deg = d0_ref[:, 0:1] + d1_ref[:, 0:1] + 1.0
        dis = lax.rsqrt(deg)
        dis_ref[...] = dis
        xw = jnp.dot(x_ref[...], w_ref[...], preferred_element_type=jnp.float32)
        y_ref[...] = dis * xw

    return pl.pallas_call(
        body,
        grid=(N // BN,),
        in_specs=[
            pl.BlockSpec((BN, D), lambda i: (i, 0)),
            pl.BlockSpec((BN, D), lambda i: (i, 0)),
            pl.BlockSpec((BN, D), lambda i: (i, 0)),
            pl.BlockSpec((D, D), lambda i: (0, 0)),
        ],
        out_specs=[
            pl.BlockSpec((BN, 1), lambda i: (i, 0)),
            pl.BlockSpec((BN, D), lambda i: (i, 0)),
        ],
        out_shape=[
            jax.ShapeDtypeStruct((N, 1), jnp.float32),
            jax.ShapeDtypeStruct((N, D), jnp.float32),
        ],
    )(degp[0], degp[1], x, W1)


def _tc_mid(p, dis, b, Wn):
    def body(p0_ref, p1_ref, dis_ref, b_ref, w_ref, y_ref):
        dis = dis_ref[...]
        h = jnp.maximum(dis * (p0_ref[...] + p1_ref[...]) + b_ref[...], 0.0)
        hw = jnp.dot(h, w_ref[...], preferred_element_type=jnp.float32)
        y_ref[...] = dis * hw

    return pl.pallas_call(
        body,
        grid=(N // BN,),
        in_specs=[
            pl.BlockSpec((BN, D), lambda i: (i, 0)),
            pl.BlockSpec((BN, D), lambda i: (i, 0)),
            pl.BlockSpec((BN, 1), lambda i: (i, 0)),
            pl.BlockSpec((1, D), lambda i: (0, 0)),
            pl.BlockSpec((D, D), lambda i: (0, 0)),
        ],
        out_specs=pl.BlockSpec((BN, D), lambda i: (i, 0)),
        out_shape=jax.ShapeDtypeStruct((N, D), jnp.float32),
    )(p[0], p[1], dis, b, Wn)


def _tc_final(p, dis, b, batch, Wr, br):
    def body(p0_ref, p1_ref, dis_ref, b_ref, bat_ref, wr_ref, br_ref,
             out_ref, pool_acc, cnt_acc):
        i = pl.program_id(0)
        dis = dis_ref[...]
        h = jnp.maximum(dis * (p0_ref[...] + p1_ref[...]) + b_ref[...], 0.0)
        seg = lax.broadcasted_iota(jnp.int32, (BN, G), 1)
        mask = jnp.where(bat_ref[...] == seg, 1.0, 0.0)
        dn = (((0,), (0,)), ((), ()))
        psum = lax.dot_general(mask, h, dn,
                               preferred_element_type=jnp.float32)
        csum = lax.dot_general(mask, jnp.ones((BN, D), jnp.float32), dn,
                               preferred_element_type=jnp.float32)

        @pl.when(i == 0)
        def _():
            pool_acc[...] = psum
            cnt_acc[...] = csum

        @pl.when(i > 0)
        def _():
            pool_acc[...] += psum
            cnt_acc[...] += csum

        @pl.when(i == N // BN - 1)
        def _():
            pooled = pool_acc[...] / jnp.maximum(cnt_acc[...], 1.0)
            out_ref[...] = jnp.dot(pooled, wr_ref[...],
                                   preferred_element_type=jnp.float32) \
                + br_ref[...]

    return pl.pallas_call(
        body,
        grid=(N // BN,),
        in_specs=[
            pl.BlockSpec((BN, D), lambda i: (i, 0)),
            pl.BlockSpec((BN, D), lambda i: (i, 0)),
            pl.BlockSpec((BN, 1), lambda i: (i, 0)),
            pl.BlockSpec((1, D), lambda i: (0, 0)),
            pl.BlockSpec((BN, 1), lambda i: (i, 0)),
            pl.BlockSpec((D, OUT_PAD), lambda i: (0, 0)),
            pl.BlockSpec((1, OUT_PAD), lambda i: (0, 0)),
        ],
        out_specs=pl.BlockSpec((G, OUT_PAD), lambda i: (0, 0)),
        out_shape=jax.ShapeDtypeStruct((G, OUT_PAD), jnp.float32),
        scratch_shapes=[
            pltpu.VMEM((G, D), jnp.float32),
            pltpu.VMEM((G, D), jnp.float32),
        ],
    )(p[0], p[1], dis, b, batch, Wr, br)


def kernel(x, batch, edge_index, W1, b1, W2, b2, W3, b3, Wr, br):
    src = edge_index[0].astype(jnp.int32)
    dst = edge_index[1].astype(jnp.int32)
    batch = batch.astype(jnp.int32).reshape(N, 1)

    srcw = jnp.pad(src.reshape(NW, EW), ((0, 0), (0, EWP - EW)),
                   constant_values=0).reshape(NC, NS, NCHUNK, CHUNK)
    dstw = jnp.pad(dst.reshape(NW, EW), ((0, 0), (0, EWP - EW)),
                   constant_values=N).reshape(NC, NS, NCHUNK, CHUNK)

    b1r = b1.reshape(1, D)
    b2r = b2.reshape(1, D)
    b3r = b3.reshape(1, D)
    Wrp = jnp.pad(Wr, ((0, 0), (0, OUT_PAD - OUT)))
    brp = jnp.pad(br, (0, OUT_PAD - OUT)).reshape(1, OUT_PAD)

    degp = _sc_deg(dstw)
    dis, y1 = _tc1(degp, x, W1)
    p = _sc_agg(y1, srcw, dstw)
    y2 = _tc_mid(p, dis, b1r, W2)
    p = _sc_agg(y2, srcw, dstw)
    y3 = _tc_mid(p, dis, b2r, W3)
    p = _sc_agg(y3, srcw, dstw)
    out = _tc_final(p, dis, b3r, batch, Wrp, brp)
    return out[:, :OUT]


# speedup vs baseline: 1.0025x; 1.0025x over previous
import functools

import jax
import jax.numpy as jnp
from jax import lax
from jax.experimental import pallas as pl
from jax.experimental.pallas import tpu as pltpu
from jax.experimental.pallas import tpu_sc as plsc

N = 10000
E = 320000
D = 128
G = 64
OUT = 2001
OUT_PAD = 2048

NC, NS = 2, 16
NW = NC * NS
EW = E // NW
CHUNK = 128
IB = 16
NSTAGE = 5
NCHUNK = NSTAGE * IB
EWP = NCHUNK * CHUNK
RPT = 640
NPAD = RPT * NS
RLAST = N - (NS - 1) * RPT
BN = 1000

_MESH = dict(core_axis_name="c", subcore_axis_name="s", num_cores=NC,
             num_subcores=NS)


def _sc_deg(dstw):
    mesh = plsc.VectorSubcoreMesh(**_MESH)

    @functools.partial(
        pl.kernel,
        out_type=jax.ShapeDtypeStruct((NC, N, D), jnp.float32),
        mesh=mesh,
        scratch_types=[
            pltpu.VMEM((NCHUNK, CHUNK), jnp.int32),
            pltpu.VMEM((CHUNK, D), jnp.float32),
            pltpu.VMEM((CHUNK, D), jnp.float32),
            pltpu.VMEM_SHARED((NPAD, D), jnp.float32),
        ],
    )
    def k(dstw_hbm, out_hbm, dst_v, ones_v, z_v, accum):
        c = lax.axis_index("c")
        s = lax.axis_index("s")
        r0 = pl.multiple_of(s * RPT, 8)

        pltpu.sync_copy(dstw_hbm.at[c, s], dst_v)

        one_row = jnp.where(lax.iota(jnp.int32, 16) == 0, 1.0, 0.0)
        zero_row = jnp.zeros((16,), jnp.float32)

        def fill_ones(i, _):
            for u in range(D // 16):
                ones_v[i, pl.ds(u * 16, 16)] = one_row if u == 0 else zero_row
                z_v[i, pl.ds(u * 16, 16)] = zero_row
            return 0

        lax.fori_loop(0, CHUNK, fill_ones, 0)

        for q in range(RPT // CHUNK):
            pltpu.sync_copy(z_v, accum.at[pl.ds(r0 + q * CHUNK, CHUNK)])

        plsc.subcore_barrier()

        def body(j, _):
            pltpu.sync_copy(ones_v, accum.at[dst_v.at[j]], add=True)
            return 0

        lax.fori_loop(0, NCHUNK, body, 0)
        plsc.subcore_barrier()

        @pl.when(s < NS - 1)
        def _():
            pltpu.sync_copy(accum.at[pl.ds(r0, RPT)],
                            out_hbm.at[c, pl.ds(r0, RPT)])

        @pl.when(s == NS - 1)
        def _():
            pltpu.sync_copy(accum.at[pl.ds((NS - 1) * RPT, RLAST)],
                            out_hbm.at[c, pl.ds((NS - 1) * RPT, RLAST)])

    return k(dstw)


def _sc_agg(y, srcw, dstw):
    mesh = plsc.VectorSubcoreMesh(**_MESH)

    @functools.partial(
        pl.kernel,
        out_type=jax.ShapeDtypeStruct((NC, N, D), jnp.float32),
        mesh=mesh,
        scratch_types=[
            pltpu.VMEM((NCHUNK, CHUNK), jnp.int32),
            pltpu.VMEM((NCHUNK, CHUNK), jnp.int32),
            pltpu.VMEM((CHUNK, D), jnp.float32),
            pltpu.VMEM_SHARED((NPAD, D), jnp.float32),
            pltpu.SemaphoreType.DMA,
        ],
    )
    def k(y_hbm, srcw_hbm, dstw_hbm, out_hbm, src_v, dst_v, gbuf, accum,
          sem_g):
        c = lax.axis_index("c")
        s = lax.axis_index("s")
        r0 = pl.multiple_of(s * RPT, 8)

        pltpu.sync_copy(srcw_hbm.at[c, s], src_v)
        pltpu.sync_copy(dstw_hbm.at[c, s], dst_v)

        @pl.when(jnp.logical_and(c == 0, s < NS - 1))
        def _():
            pltpu.sync_copy(y_hbm.at[pl.ds(r0, RPT)], accum.at[pl.ds(r0, RPT)])

        @pl.when(jnp.logical_and(c == 0, s == NS - 1))
        def _():
            pltpu.sync_copy(y_hbm.at[pl.ds((NS - 1) * RPT, RLAST)],
                            accum.at[pl.ds((NS - 1) * RPT, RLAST)])

        @pl.when(c == 1)
        def _():
            def fill_zeros(i, _):
                for u in range(D // 16):
                    gbuf[i, pl.ds(u * 16, 16)] = jnp.zeros((16,),
                                                           jnp.float32)
                return 0

            lax.fori_loop(0, CHUNK, fill_zeros, 0)
            for q in range(RPT // CHUNK):
                pltpu.sync_copy(gbuf,
                                accum.at[pl.ds(r0 + q * CHUNK, CHUNK)])

        plsc.subcore_barrier()

        def body(j, _):
            pltpu.async_copy(y_hbm.at[src_v.at[j]], gbuf, sem_g).wait()
            pltpu.sync_copy(gbuf, accum.at[dst_v.at[j]], add=True)
            return 0

        lax.fori_loop(0, NCHUNK, body, 0)
        plsc.subcore_barrier()

        @pl.when(s < NS - 1)
        def _():
            pltpu.sync_copy(accum.at[pl.ds(r0, RPT)],
                            out_hbm.at[c, pl.ds(r0, RPT)])

        @pl.when(s == NS - 1)
        def _():
            pltpu.sync_copy(accum.at[pl.ds((NS - 1) * RPT, RLAST)],
                            out_hbm.at[c, pl.ds((NS - 1) * RPT, RLAST)])

    return k(y, srcw, dstw)


def _tc1(degp, x, W1):
    def body(d0_ref, d1_ref, x_ref, w_ref, dis_ref, y_ref):
        deg = d0_ref[:, 0:1] + d1_ref[:, 0:1] + 1.0
        dis = lax.rsqrt(deg)
        dis_ref[...] = dis
        xw = jnp.dot(x_ref[...], w_ref[...], preferred_element_type=jnp.float32)
        y_ref[...] = dis * xw

    return pl.pallas_call(
        body,
        grid=(N // BN,),
        in_specs=[
            pl.BlockSpec((BN, D), lambda i: (i, 0)),
            pl.BlockSpec((BN, D), lambda i: (i, 0)),
            pl.BlockSpec((BN, D), lambda i: (i, 0)),
            pl.BlockSpec((D, D), lambda i: (0, 0)),
        ],
        out_specs=[
            pl.BlockSpec((BN, 1), lambda i: (i, 0)),
            pl.BlockSpec((BN, D), lambda i: (i, 0)),
        ],
        out_shape=[
            jax.ShapeDtypeStruct((N, 1), jnp.float32),
            jax.ShapeDtypeStruct((N, D), jnp.float32),
        ],
    )(degp[0], degp[1], x, W1)


def _tc_mid(p, dis, b, Wn):
    def body(p0_ref, p1_ref, dis_ref, b_ref, w_ref, y_ref):
        dis = dis_ref[...]
        h = jnp.maximum(dis * (p0_ref[...] + p1_ref[...]) + b_ref[...], 0.0)
        hw = jnp.dot(h, w_ref[...], preferred_element_type=jnp.float32)
        y_ref[...] = dis * hw

    return pl.pallas_call(
        body,
        grid=(N // BN,),
        in_specs=[
            pl.BlockSpec((BN, D), lambda i: (i, 0)),
            pl.BlockSpec((BN, D), lambda i: (i, 0)),
            pl.BlockSpec((BN, 1), lambda i: (i, 0)),
            pl.BlockSpec((1, D), lambda i: (0, 0)),
            pl.BlockSpec((D, D), lambda i: (0, 0)),
        ],
        out_specs=pl.BlockSpec((BN, D), lambda i: (i, 0)),
        out_shape=jax.ShapeDtypeStruct((N, D), jnp.float32),
    )(p[0], p[1], dis, b, Wn)


def _tc_final(p, dis, b, batch, Wr, br):
    def body(p0_ref, p1_ref, dis_ref, b_ref, bat_ref, wr_ref, br_ref,
             out_ref, pool_acc, cnt_acc):
        i = pl.program_id(0)
        dis = dis_ref[...]
        h = jnp.maximum(dis * (p0_ref[...] + p1_ref[...]) + b_ref[...], 0.0)
        seg = lax.broadcasted_iota(jnp.int32, (BN, G), 1)
        mask = jnp.where(bat_ref[...] == seg, 1.0, 0.0)
        dn = (((0,), (0,)), ((), ()))
        psum = lax.dot_general(mask, h, dn,
                               preferred_element_type=jnp.float32)
        csum = lax.dot_general(mask, jnp.ones((BN, D), jnp.float32), dn,
                               preferred_element_type=jnp.float32)

        @pl.when(i == 0)
        def _():
            pool_acc[...] = psum
            cnt_acc[...] = csum

        @pl.when(i > 0)
        def _():
            pool_acc[...] += psum
            cnt_acc[...] += csum

        @pl.when(i == N // BN - 1)
        def _():
            pooled = pool_acc[...] / jnp.maximum(cnt_acc[...], 1.0)
            out_ref[...] = jnp.dot(pooled, wr_ref[...],
                                   preferred_element_type=jnp.float32) \
                + br_ref[...]

    return pl.pallas_call(
        body,
        grid=(N // BN,),
        in_specs=[
            pl.BlockSpec((BN, D), lambda i: (i, 0)),
            pl.BlockSpec((BN, D), lambda i: (i, 0)),
            pl.BlockSpec((BN, 1), lambda i: (i, 0)),
            pl.BlockSpec((1, D), lambda i: (0, 0)),
            pl.BlockSpec((BN, 1), lambda i: (i, 0)),
            pl.BlockSpec((D, OUT_PAD), lambda i: (0, 0)),
            pl.BlockSpec((1, OUT_PAD), lambda i: (0, 0)),
        ],
        out_specs=pl.BlockSpec((G, OUT_PAD), lambda i: (0, 0)),
        out_shape=jax.ShapeDtypeStruct((G, OUT_PAD), jnp.float32),
        scratch_shapes=[
            pltpu.VMEM((G, D), jnp.float32),
            pltpu.VMEM((G, D), jnp.float32),
        ],
    )(p[0], p[1], dis, b, batch, Wr, br)


def kernel(x, batch, edge_index, W1, b1, W2, b2, W3, b3, Wr, br):
    src = edge_index[0].astype(jnp.int32)
    dst = edge_index[1].astype(jnp.int32)
    batch = batch.astype(jnp.int32).reshape(N, 1)

    sink = jnp.broadcast_to(
        jnp.arange(EWP - EW, dtype=jnp.int32)[None, :] + N, (NW, EWP - EW))
    srcw = jnp.pad(src.reshape(NW, EW), ((0, 0), (0, EWP - EW)),
                   constant_values=0).reshape(NC, NS, NCHUNK, CHUNK)
    dstw = jnp.concatenate([dst.reshape(NW, EW), sink],
                           axis=1).reshape(NC, NS, NCHUNK, CHUNK)

    b1r = b1.reshape(1, D)
    b2r = b2.reshape(1, D)
    b3r = b3.reshape(1, D)
    Wrp = jnp.pad(Wr, ((0, 0), (0, OUT_PAD - OUT)))
    brp = jnp.pad(br, (0, OUT_PAD - OUT)).reshape(1, OUT_PAD)

    degp = _sc_deg(dstw)
    dis, y1 = _tc1(degp, x, W1)
    p = _sc_agg(y1, srcw, dstw)
    y2 = _tc_mid(p, dis, b1r, W2)
    p = _sc_agg(y2, srcw, dstw)
    y3 = _tc_mid(p, dis, b2r, W3)
    p = _sc_agg(y3, srcw, dstw)
    out = _tc_final(p, dis, b3r, batch, Wrp, brp)
    return out[:, :OUT]


# speedup vs baseline: 2.1828x; 2.1772x over previous
import functools

import jax
import jax.numpy as jnp
from jax import lax
from jax.experimental import pallas as pl
from jax.experimental.pallas import tpu as pltpu
from jax.experimental.pallas import tpu_sc as plsc

N = 10000
E = 320000
D = 128
G = 64
OUT = 2001
OUT_PAD = 2048

NC, NS = 2, 16
NW = NC * NS
EW = E // NW
CHUNK = 128
IB = 16
NSTAGE = 5
NCHUNK = NSTAGE * IB
EWP = NCHUNK * CHUNK
RPT = 640
NPAD = RPT * NS
RLAST = N - (NS - 1) * RPT
BN = 1000

_MESH = dict(core_axis_name="c", subcore_axis_name="s", num_cores=NC,
             num_subcores=NS)


def _sc_deg(dstw):
    mesh = plsc.VectorSubcoreMesh(**_MESH)

    @functools.partial(
        pl.kernel,
        out_type=jax.ShapeDtypeStruct((NC, N, D), jnp.float32),
        mesh=mesh,
        scratch_types=[
            pltpu.VMEM((NCHUNK, CHUNK), jnp.int32),
            pltpu.VMEM((CHUNK, D), jnp.float32),
            pltpu.VMEM((CHUNK, D), jnp.float32),
            pltpu.VMEM_SHARED((NPAD, D), jnp.float32),
        ],
    )
    def k(dstw_hbm, out_hbm, dst_v, ones_v, z_v, accum):
        c = lax.axis_index("c")
        s = lax.axis_index("s")
        r0 = pl.multiple_of(s * RPT, 8)

        pltpu.sync_copy(dstw_hbm.at[c, s], dst_v)

        one_row = jnp.where(lax.iota(jnp.int32, 16) == 0, 1.0, 0.0)
        zero_row = jnp.zeros((16,), jnp.float32)

        def fill_ones(i, _):
            for u in range(D // 16):
                ones_v[i, pl.ds(u * 16, 16)] = one_row if u == 0 else zero_row
                z_v[i, pl.ds(u * 16, 16)] = zero_row
            return 0

        lax.fori_loop(0, CHUNK, fill_ones, 0)

        for q in range(RPT // CHUNK):
            pltpu.sync_copy(z_v, accum.at[pl.ds(r0 + q * CHUNK, CHUNK)])

        plsc.subcore_barrier()

        def body(j, _):
            pltpu.sync_copy(ones_v, accum.at[dst_v.at[j]], add=True)
            return 0

        lax.fori_loop(0, NCHUNK, body, 0)
        plsc.subcore_barrier()

        @pl.when(s < NS - 1)
        def _():
            pltpu.sync_copy(accum.at[pl.ds(r0, RPT)],
                            out_hbm.at[c, pl.ds(r0, RPT)])

        @pl.when(s == NS - 1)
        def _():
            pltpu.sync_copy(accum.at[pl.ds((NS - 1) * RPT, RLAST)],
                            out_hbm.at[c, pl.ds((NS - 1) * RPT, RLAST)])

    return k(dstw)


def _sc_agg(y, srcw, dstw):
    mesh = plsc.VectorSubcoreMesh(**_MESH)

    @functools.partial(
        pl.kernel,
        out_type=jax.ShapeDtypeStruct((NC, N, D), jnp.float32),
        mesh=mesh,
        scratch_types=[
            pltpu.VMEM((NCHUNK, CHUNK), jnp.int32),
            pltpu.VMEM((NCHUNK, CHUNK), jnp.int32),
            pltpu.VMEM((CHUNK, D), jnp.float32),
            pltpu.VMEM_SHARED((NPAD, D), jnp.float32),
            pltpu.SemaphoreType.DMA,
        ],
    )
    def k(y_hbm, srcw_hbm, dstw_hbm, out_hbm, src_v, dst_v, gbuf, accum,
          sem_g):
        c = lax.axis_index("c")
        s = lax.axis_index("s")
        r0 = pl.multiple_of(s * RPT, 8)

        pltpu.sync_copy(srcw_hbm.at[c, s], src_v)
        pltpu.sync_copy(dstw_hbm.at[c, s], dst_v)

        @pl.when(jnp.logical_and(c == 0, s < NS - 1))
        def _():
            pltpu.sync_copy(y_hbm.at[pl.ds(r0, RPT)], accum.at[pl.ds(r0, RPT)])

        @pl.when(jnp.logical_and(c == 0, s == NS - 1))
        def _():
            pltpu.sync_copy(y_hbm.at[pl.ds((NS - 1) * RPT, RLAST)],
                            accum.at[pl.ds((NS - 1) * RPT, RLAST)])

        @pl.when(c == 1)
        def _():
            def fill_zeros(i, _):
                for u in range(D // 16):
                    gbuf[i, pl.ds(u * 16, 16)] = jnp.zeros((16,),
                                                           jnp.float32)
                return 0

            lax.fori_loop(0, CHUNK, fill_zeros, 0)
            for q in range(RPT // CHUNK):
                pltpu.sync_copy(gbuf,
                                accum.at[pl.ds(r0 + q * CHUNK, CHUNK)])

        plsc.subcore_barrier()

        def body(j, _):
            pltpu.async_copy(y_hbm.at[src_v.at[j]], gbuf, sem_g).wait()
            pltpu.sync_copy(gbuf, accum.at[dst_v.at[j]], add=True)
            return 0

        lax.fori_loop(0, NCHUNK, body, 0)
        plsc.subcore_barrier()

        @pl.when(s < NS - 1)
        def _():
            pltpu.sync_copy(accum.at[pl.ds(r0, RPT)],
                            out_hbm.at[c, pl.ds(r0, RPT)])

        @pl.when(s == NS - 1)
        def _():
            pltpu.sync_copy(accum.at[pl.ds((NS - 1) * RPT, RLAST)],
                            out_hbm.at[c, pl.ds((NS - 1) * RPT, RLAST)])

    return k(y, srcw, dstw)


def _tc1(degp, x, W1):
    def body(d0_ref, d1_ref, x_ref, w_ref, dis_ref, y_ref):
        deg = d0_ref[:, 0:1] + d1_ref[:, 0:1] + 1.0
        dis = lax.rsqrt(deg)
        dis_ref[...] = dis
        xw = jnp.dot(x_ref[...], w_ref[...], preferred_element_type=jnp.float32)
        y_ref[...] = dis * xw

    return pl.pallas_call(
        body,
        grid=(N // BN,),
        in_specs=[
            pl.BlockSpec((BN, D), lambda i: (i, 0)),
            pl.BlockSpec((BN, D), lambda i: (i, 0)),
            pl.BlockSpec((BN, D), lambda i: (i, 0)),
            pl.BlockSpec((D, D), lambda i: (0, 0)),
        ],
        out_specs=[
            pl.BlockSpec((BN, 1), lambda i: (i, 0)),
            pl.BlockSpec((BN, D), lambda i: (i, 0)),
        ],
        out_shape=[
            jax.ShapeDtypeStruct((N, 1), jnp.float32),
            jax.ShapeDtypeStruct((N, D), jnp.float32),
        ],
    )(degp[0], degp[1], x, W1)


def _tc_mid(p, dis, b, Wn):
    def body(p0_ref, p1_ref, dis_ref, b_ref, w_ref, y_ref):
        dis = dis_ref[...]
        h = jnp.maximum(dis * (p0_ref[...] + p1_ref[...]) + b_ref[...], 0.0)
        hw = jnp.dot(h, w_ref[...], preferred_element_type=jnp.float32)
        y_ref[...] = dis * hw

    return pl.pallas_call(
        body,
        grid=(N // BN,),
        in_specs=[
            pl.BlockSpec((BN, D), lambda i: (i, 0)),
            pl.BlockSpec((BN, D), lambda i: (i, 0)),
            pl.BlockSpec((BN, 1), lambda i: (i, 0)),
            pl.BlockSpec((1, D), lambda i: (0, 0)),
            pl.BlockSpec((D, D), lambda i: (0, 0)),
        ],
        out_specs=pl.BlockSpec((BN, D), lambda i: (i, 0)),
        out_shape=jax.ShapeDtypeStruct((N, D), jnp.float32),
    )(p[0], p[1], dis, b, Wn)


def _tc_final(p, dis, b, batch, Wr, br):
    def body(p0_ref, p1_ref, dis_ref, b_ref, bat_ref, wr_ref, br_ref,
             out_ref, pool_acc, cnt_acc):
        i = pl.program_id(0)
        dis = dis_ref[...]
        h = jnp.maximum(dis * (p0_ref[...] + p1_ref[...]) + b_ref[...], 0.0)
        seg = lax.broadcasted_iota(jnp.int32, (BN, G), 1)
        mask = jnp.where(bat_ref[...] == seg, 1.0, 0.0)
        dn = (((0,), (0,)), ((), ()))
        psum = lax.dot_general(mask, h, dn,
                               preferred_element_type=jnp.float32)
        csum = lax.dot_general(mask, jnp.ones((BN, D), jnp.float32), dn,
                               preferred_element_type=jnp.float32)

        @pl.when(i == 0)
        def _():
            pool_acc[...] = psum
            cnt_acc[...] = csum

        @pl.when(i > 0)
        def _():
            pool_acc[...] += psum
            cnt_acc[...] += csum

        @pl.when(i == N // BN - 1)
        def _():
            pooled = pool_acc[...] / jnp.maximum(cnt_acc[...], 1.0)
            out_ref[...] = jnp.dot(pooled, wr_ref[...],
                                   preferred_element_type=jnp.float32) \
                + br_ref[...]

    return pl.pallas_call(
        body,
        grid=(N // BN,),
        in_specs=[
            pl.BlockSpec((BN, D), lambda i: (i, 0)),
            pl.BlockSpec((BN, D), lambda i: (i, 0)),
            pl.BlockSpec((BN, 1), lambda i: (i, 0)),
            pl.BlockSpec((1, D), lambda i: (0, 0)),
            pl.BlockSpec((BN, 1), lambda i: (i, 0)),
            pl.BlockSpec((D, OUT_PAD), lambda i: (0, 0)),
            pl.BlockSpec((1, OUT_PAD), lambda i: (0, 0)),
        ],
        out_specs=pl.BlockSpec((G, OUT_PAD), lambda i: (0, 0)),
        out_shape=jax.ShapeDtypeStruct((G, OUT_PAD), jnp.float32),
        scratch_shapes=[
            pltpu.VMEM((G, D), jnp.float32),
            pltpu.VMEM((G, D), jnp.float32),
        ],
    )(p[0], p[1], dis, b, batch, Wr, br)


def kernel(x, batch, edge_index, W1, b1, W2, b2, W3, b3, Wr, br):
    src = edge_index[0].astype(jnp.int32)
    dst = edge_index[1].astype(jnp.int32)
    batch = batch.astype(jnp.int32).reshape(N, 1)

    pad_ids = jnp.arange(EWP - EW, dtype=jnp.int32)[None, :]
    sink = jnp.broadcast_to(pad_ids + N, (NW, EWP - EW))
    psrc = jnp.broadcast_to(pad_ids, (NW, EWP - EW))
    srcw = jnp.concatenate([src.reshape(NW, EW), psrc],
                           axis=1).reshape(NC, NS, NCHUNK, CHUNK)
    dstw = jnp.concatenate([dst.reshape(NW, EW), sink],
                           axis=1).reshape(NC, NS, NCHUNK, CHUNK)

    b1r = b1.reshape(1, D)
    b2r = b2.reshape(1, D)
    b3r = b3.reshape(1, D)
    Wrp = jnp.pad(Wr, ((0, 0), (0, OUT_PAD - OUT)))
    brp = jnp.pad(br, (0, OUT_PAD - OUT)).reshape(1, OUT_PAD)

    degp = _sc_deg(dstw)
    dis, y1 = _tc1(degp, x, W1)
    p = _sc_agg(y1, srcw, dstw)
    y2 = _tc_mid(p, dis, b1r, W2)
    p = _sc_agg(y2, srcw, dstw)
    y3 = _tc_mid(p, dis, b2r, W3)
    p = _sc_agg(y3, srcw, dstw)
    out = _tc_final(p, dis, b3r, batch, Wrp, brp)
    return out[:, :OUT]


# speedup vs baseline: 2.1840x; 1.0006x over previous
import functools

import jax
import jax.numpy as jnp
from jax import lax
from jax.experimental import pallas as pl
from jax.experimental.pallas import tpu as pltpu
from jax.experimental.pallas import tpu_sc as plsc

N = 10000
E = 320000
D = 128
G = 64
OUT = 2001
OUT_PAD = 2048

NC, NS = 2, 16
NW = NC * NS
EW = E // NW
CHUNK = 128
NCHUNK = 80
EWP = NCHUNK * CHUNK
RPT = 640
NPAD = RPT * NS
RLAST = N - (NS - 1) * RPT
BN = 1000

_MESH = dict(core_axis_name="c", subcore_axis_name="s", num_cores=NC,
             num_subcores=NS)


def _sc_deg(dstw):
    mesh = plsc.VectorSubcoreMesh(**_MESH)

    @functools.partial(
        pl.kernel,
        out_type=jax.ShapeDtypeStruct((NC, N, D), jnp.float32),
        mesh=mesh,
        scratch_types=[
            pltpu.VMEM((NCHUNK, CHUNK), jnp.int32),
            pltpu.VMEM((CHUNK, D), jnp.float32),
            pltpu.VMEM((CHUNK, D), jnp.float32),
            pltpu.VMEM_SHARED((NPAD, D), jnp.float32),
        ],
    )
    def k(dstw_hbm, out_hbm, dst_v, ones_v, z_v, accum):
        c = lax.axis_index("c")
        s = lax.axis_index("s")
        r0 = pl.multiple_of(s * RPT, 8)

        pltpu.sync_copy(dstw_hbm.at[c, s], dst_v)

        one_row = jnp.where(lax.iota(jnp.int32, 16) == 0, 1.0, 0.0)
        zero_row = jnp.zeros((16,), jnp.float32)

        def fill_ones(i, _):
            for u in range(D // 16):
                ones_v[i, pl.ds(u * 16, 16)] = one_row if u == 0 else zero_row
                z_v[i, pl.ds(u * 16, 16)] = zero_row
            return 0

        lax.fori_loop(0, CHUNK, fill_ones, 0)

        for q in range(RPT // CHUNK):
            pltpu.sync_copy(z_v, accum.at[pl.ds(r0 + q * CHUNK, CHUNK)])

        plsc.subcore_barrier()

        def body(j, _):
            pltpu.sync_copy(ones_v, accum.at[dst_v.at[j]], add=True)
            return 0

        lax.fori_loop(0, NCHUNK, body, 0)
        plsc.subcore_barrier()

        @pl.when(s < NS - 1)
        def _():
            pltpu.sync_copy(accum.at[pl.ds(r0, RPT)],
                            out_hbm.at[c, pl.ds(r0, RPT)])

        @pl.when(s == NS - 1)
        def _():
            pltpu.sync_copy(accum.at[pl.ds((NS - 1) * RPT, RLAST)],
                            out_hbm.at[c, pl.ds((NS - 1) * RPT, RLAST)])

    return k(dstw)


def _sc_agg(y, srcw, dstw):
    mesh = plsc.VectorSubcoreMesh(**_MESH)

    @functools.partial(
        pl.kernel,
        out_type=jax.ShapeDtypeStruct((NC, N, D), jnp.float32),
        mesh=mesh,
        scratch_types=[
            pltpu.VMEM((NCHUNK, CHUNK), jnp.int32),
            pltpu.VMEM((NCHUNK, CHUNK), jnp.int32),
            pltpu.VMEM((CHUNK, D), jnp.float32),
            pltpu.VMEM_SHARED((NPAD, D), jnp.float32),
            pltpu.SemaphoreType.DMA,
        ],
    )
    def k(y_hbm, srcw_hbm, dstw_hbm, out_hbm, src_v, dst_v, gbuf, accum,
          sem_g):
        c = lax.axis_index("c")
        s = lax.axis_index("s")
        r0 = pl.multiple_of(s * RPT, 8)

        pltpu.sync_copy(srcw_hbm.at[c, s], src_v)
        pltpu.sync_copy(dstw_hbm.at[c, s], dst_v)

        @pl.when(jnp.logical_and(c == 0, s < NS - 1))
        def _():
            pltpu.sync_copy(y_hbm.at[pl.ds(r0, RPT)], accum.at[pl.ds(r0, RPT)])

        @pl.when(jnp.logical_and(c == 0, s == NS - 1))
        def _():
            pltpu.sync_copy(y_hbm.at[pl.ds((NS - 1) * RPT, RLAST)],
                            accum.at[pl.ds((NS - 1) * RPT, RLAST)])

        @pl.when(c == 1)
        def _():
            def fill_zeros(i, _):
                for u in range(D // 16):
                    gbuf[i, pl.ds(u * 16, 16)] = jnp.zeros((16,),
                                                           jnp.float32)
                return 0

            lax.fori_loop(0, CHUNK, fill_zeros, 0)
            for q in range(RPT // CHUNK):
                pltpu.sync_copy(gbuf,
                                accum.at[pl.ds(r0 + q * CHUNK, CHUNK)])

        plsc.subcore_barrier()

        def body(j, _):
            pltpu.async_copy(y_hbm.at[src_v.at[j]], gbuf, sem_g).wait()
            pltpu.sync_copy(gbuf, accum.at[dst_v.at[j]], add=True)
            return 0

        lax.fori_loop(0, NCHUNK, body, 0)
        plsc.subcore_barrier()

        @pl.when(s < NS - 1)
        def _():
            pltpu.sync_copy(accum.at[pl.ds(r0, RPT)],
                            out_hbm.at[c, pl.ds(r0, RPT)])

        @pl.when(s == NS - 1)
        def _():
            pltpu.sync_copy(accum.at[pl.ds((NS - 1) * RPT, RLAST)],
                            out_hbm.at[c, pl.ds((NS - 1) * RPT, RLAST)])

    return k(y, srcw, dstw)


def _tc1(degp, x, W1):
    def body(d0_ref, d1_ref, x_ref, w_ref, dis_ref, y_ref):
        deg = d0_ref[:, 0:1] + d1_ref[:, 0:1] + 1.0
        dis = lax.rsqrt(deg)
        dis_ref[...] = dis
        xw = jnp.dot(x_ref[...], w_ref[...], preferred_element_type=jnp.float32)
        y_ref[...] = dis * xw

    return pl.pallas_call(
        body,
        grid=(N // BN,),
        in_specs=[
            pl.BlockSpec((BN, D), lambda i: (i, 0)),
            pl.BlockSpec((BN, D), lambda i: (i, 0)),
            pl.BlockSpec((BN, D), lambda i: (i, 0)),
            pl.BlockSpec((D, D), lambda i: (0, 0)),
        ],
        out_specs=[
            pl.BlockSpec((BN, 1), lambda i: (i, 0)),
            pl.BlockSpec((BN, D), lambda i: (i, 0)),
        ],
        out_shape=[
            jax.ShapeDtypeStruct((N, 1), jnp.float32),
            jax.ShapeDtypeStruct((N, D), jnp.float32),
        ],
    )(degp[0], degp[1], x, W1)


def _tc_mid(p, dis, b, Wn):
    def body(p0_ref, p1_ref, dis_ref, b_ref, w_ref, y_ref):
        dis = dis_ref[...]
        h = jnp.maximum(dis * (p0_ref[...] + p1_ref[...]) + b_ref[...], 0.0)
        hw = jnp.dot(h, w_ref[...], preferred_element_type=jnp.float32)
        y_ref[...] = dis * hw

    return pl.pallas_call(
        body,
        grid=(N // BN,),
        in_specs=[
            pl.BlockSpec((BN, D), lambda i: (i, 0)),
            pl.BlockSpec((BN, D), lambda i: (i, 0)),
            pl.BlockSpec((BN, 1), lambda i: (i, 0)),
            pl.BlockSpec((1, D), lambda i: (0, 0)),
            pl.BlockSpec((D, D), lambda i: (0, 0)),
        ],
        out_specs=pl.BlockSpec((BN, D), lambda i: (i, 0)),
        out_shape=jax.ShapeDtypeStruct((N, D), jnp.float32),
    )(p[0], p[1], dis, b, Wn)


def _tc_final(p, dis, b, batch, Wr, br):
    def body(p0_ref, p1_ref, dis_ref, b_ref, bat_ref, wr_ref, br_ref,
             out_ref, pool_acc, cnt_acc):
        i = pl.program_id(0)
        dis = dis_ref[...]
        h = jnp.maximum(dis * (p0_ref[...] + p1_ref[...]) + b_ref[...], 0.0)
        seg = lax.broadcasted_iota(jnp.int32, (BN, G), 1)
        mask = jnp.where(bat_ref[...] == seg, 1.0, 0.0)
        dn = (((0,), (0,)), ((), ()))
        psum = lax.dot_general(mask, h, dn,
                               preferred_element_type=jnp.float32)
        csum = lax.dot_general(mask, jnp.ones((BN, D), jnp.float32), dn,
                               preferred_element_type=jnp.float32)

        @pl.when(i == 0)
        def _():
            pool_acc[...] = psum
            cnt_acc[...] = csum

        @pl.when(i > 0)
        def _():
            pool_acc[...] += psum
            cnt_acc[...] += csum

        @pl.when(i == N // BN - 1)
        def _():
            pooled = pool_acc[...] / jnp.maximum(cnt_acc[...], 1.0)
            out_ref[...] = jnp.dot(pooled, wr_ref[...],
                                   preferred_element_type=jnp.float32) \
                + br_ref[...]

    return pl.pallas_call(
        body,
        grid=(N // BN,),
        in_specs=[
            pl.BlockSpec((BN, D), lambda i: (i, 0)),
            pl.BlockSpec((BN, D), lambda i: (i, 0)),
            pl.BlockSpec((BN, 1), lambda i: (i, 0)),
            pl.BlockSpec((1, D), lambda i: (0, 0)),
            pl.BlockSpec((BN, 1), lambda i: (i, 0)),
            pl.BlockSpec((D, OUT_PAD), lambda i: (0, 0)),
            pl.BlockSpec((1, OUT_PAD), lambda i: (0, 0)),
        ],
        out_specs=pl.BlockSpec((G, OUT_PAD), lambda i: (0, 0)),
        out_shape=jax.ShapeDtypeStruct((G, OUT_PAD), jnp.float32),
        scratch_shapes=[
            pltpu.VMEM((G, D), jnp.float32),
            pltpu.VMEM((G, D), jnp.float32),
        ],
    )(p[0], p[1], dis, b, batch, Wr, br)


def kernel(x, batch, edge_index, W1, b1, W2, b2, W3, b3, Wr, br):
    src = edge_index[0].astype(jnp.int32)
    dst = edge_index[1].astype(jnp.int32)
    batch = batch.astype(jnp.int32).reshape(N, 1)

    pad_ids = jnp.arange(EWP - EW, dtype=jnp.int32)[None, :]
    sink = jnp.broadcast_to(pad_ids + N, (NW, EWP - EW))
    psrc = jnp.broadcast_to(pad_ids, (NW, EWP - EW))
    srcw = jnp.concatenate([src.reshape(NW, EW), psrc],
                           axis=1).reshape(NC, NS, NCHUNK, CHUNK)
    dstw = jnp.concatenate([dst.reshape(NW, EW), sink],
                           axis=1).reshape(NC, NS, NCHUNK, CHUNK)

    b1r = b1.reshape(1, D)
    b2r = b2.reshape(1, D)
    b3r = b3.reshape(1, D)
    Wrp = jnp.pad(Wr, ((0, 0), (0, OUT_PAD - OUT)))
    brp = jnp.pad(br, (0, OUT_PAD - OUT)).reshape(1, OUT_PAD)

    degp = _sc_deg(dstw)
    dis, y1 = _tc1(degp, x, W1)
    p = _sc_agg(y1, srcw, dstw)
    y2 = _tc_mid(p, dis, b1r, W2)
    p = _sc_agg(y2, srcw, dstw)
    y3 = _tc_mid(p, dis, b2r, W3)
    p = _sc_agg(y3, srcw, dstw)
    out = _tc_final(p, dis, b3r, batch, Wrp, brp)
    return out[:, :OUT]
